# baseline (speedup 1.0000x reference)
import functools

import jax
import jax.numpy as jnp
from jax import lax
from jax.experimental import pallas as pl
from jax.experimental.pallas import tpu as pltpu
from jax.experimental.pallas import tpu_sc as plsc

B, N, D, E = 16, 2048, 256, 128
NO = N + E
NC, NS, L = 2, 16, 16
NW = NC * NS
PW = (B * E) // NW
WPB = E // PW
RW = N // WPB

_mesh = plsc.VectorSubcoreMesh(
    core_axis_name="c", subcore_axis_name="s", num_cores=NC, num_subcores=NS
)


@functools.partial(
    pl.kernel,
    out_type=jax.ShapeDtypeStruct((B * NO, D), jnp.float32),
    mesh=_mesh,
    scratch_types=[
        pltpu.VMEM((PW,), jnp.int32),
        pltpu.VMEM((PW,), jnp.int32),
        pltpu.VMEM((PW,), jnp.int32),
        pltpu.VMEM((PW,), jnp.int32),
        pltpu.VMEM((PW, L), jnp.float32),
        pltpu.VMEM((PW, D), jnp.float32),
        pltpu.VMEM((PW, D), jnp.float32),
        pltpu.VMEM((PW, D), jnp.float32),
        pltpu.SemaphoreType.DMA,
        pltpu.SemaphoreType.DMA,
        pltpu.SemaphoreType.DMA,
    ],
)
def _sc_unpool(table, idx0, idx1, pos, out,
               idx0_v, idx1_v, g0_v, g1_v, p_v, r0_v, r1_v, o_v,
               semc, sem0, sem1):
    wid = lax.axis_index("s") * NC + lax.axis_index("c")
    b = wid // WPB
    h = wid % WPB
    e0 = h * PW

    src = b * N + h * RW
    dst = b * NO + h * RW
    cpc = pltpu.async_copy(table.at[pl.ds(src, RW)], out.at[pl.ds(dst, RW)],
                           semc)

    pltpu.sync_copy(idx0.at[pl.ds(e0, PW)], idx0_v)
    pltpu.sync_copy(idx1.at[pl.ds(e0, PW)], idx1_v)
    pltpu.sync_copy(pos.at[pl.ds(e0, PW)], p_v)

    off = b * N
    for k in range(PW // L):
        sl = pl.ds(k * L, L)
        g0_v[sl] = idx0_v[sl] + off
        g1_v[sl] = idx1_v[sl] + off

    cp0 = pltpu.async_copy(table.at[g0_v], r0_v, sem0)
    cp1 = pltpu.async_copy(table.at[g1_v], r1_v, sem1)
    cp0.wait()
    cp1.wait()

    def row(j, carry):
        pj = p_v[j, :]
        qj = 1.0 - pj
        for k in range(D // L):
            sl = pl.ds(k * L, L)
            o_v[j, sl] = r0_v[j, sl] * pj + r1_v[j, sl] * qj
        return carry

    lax.fori_loop(0, PW, row, 0)
    pltpu.sync_copy(o_v, out.at[pl.ds(b * NO + N + e0, PW)])
    cpc.wait()


def kernel(inputs, new_pts_pos, unpool_idx):
    idx = unpool_idx.astype(jnp.int32)
    table = inputs.reshape(B * N, D)
    pos_rep = jnp.broadcast_to(new_pts_pos[:, None], (E, L))
    out = _sc_unpool(table, idx[:, 0], idx[:, 1], pos_rep)
    return out.reshape(B, NO, D)


# speedup vs baseline: 21.2542x; 21.2542x over previous
import functools

import jax
import jax.numpy as jnp
from jax import lax
from jax.experimental import pallas as pl
from jax.experimental.pallas import tpu as pltpu
from jax.experimental.pallas import tpu_sc as plsc

B, N, D, E = 16, 2048, 256, 128
NO = N + E
NC, NS, L = 2, 16, 16
NW = NC * NS
PW = (B * E) // NW
WPB = E // PW

_mesh = plsc.VectorSubcoreMesh(
    core_axis_name="c", subcore_axis_name="s", num_cores=NC, num_subcores=NS
)


@functools.partial(
    pl.kernel,
    out_type=jax.ShapeDtypeStruct((B * E, D), jnp.float32),
    mesh=_mesh,
    scratch_types=[
        pltpu.VMEM((PW,), jnp.int32),
        pltpu.VMEM((PW,), jnp.int32),
        pltpu.VMEM((PW,), jnp.int32),
        pltpu.VMEM((PW,), jnp.int32),
        pltpu.VMEM((PW, L), jnp.float32),
        pltpu.VMEM((PW, D), jnp.float32),
        pltpu.VMEM((PW, D), jnp.float32),
        pltpu.VMEM((PW, D), jnp.float32),
        pltpu.SemaphoreType.DMA,
        pltpu.SemaphoreType.DMA,
    ],
)
def _sc_edge_lerp(table, idx0, idx1, pos, out,
                  idx0_v, idx1_v, g0_v, g1_v, p_v, r0_v, r1_v, o_v,
                  sem0, sem1):
    wid = lax.axis_index("s") * NC + lax.axis_index("c")
    b = wid // WPB
    e0 = (wid % WPB) * PW
    base = wid * PW

    pltpu.sync_copy(idx0.at[pl.ds(e0, PW)], idx0_v)
    pltpu.sync_copy(idx1.at[pl.ds(e0, PW)], idx1_v)
    pltpu.sync_copy(pos.at[pl.ds(e0, PW)], p_v)

    off = b * N
    for k in range(PW // L):
        sl = pl.ds(k * L, L)
        g0_v[sl] = idx0_v[sl] + off
        g1_v[sl] = idx1_v[sl] + off

    cp0 = pltpu.async_copy(table.at[g0_v], r0_v, sem0)
    cp1 = pltpu.async_copy(table.at[g1_v], r1_v, sem1)
    cp0.wait()
    cp1.wait()

    def row(j, carry):
        pj = p_v[j, :]
        qj = 1.0 - pj
        for k in range(D // L):
            sl = pl.ds(k * L, L)
            o_v[j, sl] = r0_v[j, sl] * pj + r1_v[j, sl] * qj
        return carry

    lax.fori_loop(0, PW, row, 0)
    pltpu.sync_copy(o_v, out.at[pl.ds(base, PW)])


def _tc_copy_body(in_ref, out_ref):
    out_ref[:, :N, :] = in_ref[...]


def kernel(inputs, new_pts_pos, unpool_idx):
    idx = unpool_idx.astype(jnp.int32)
    table = inputs.reshape(B * N, D)
    pos_rep = jnp.broadcast_to(new_pts_pos[:, None], (E, L))
    edges = _sc_edge_lerp(table, idx[:, 0], idx[:, 1], pos_rep)
    edges = edges.reshape(B, E, D)
    main = pl.pallas_call(
        _tc_copy_body,
        grid=(B,),
        in_specs=[pl.BlockSpec((1, N, D), lambda i: (i, 0, 0))],
        out_specs=pl.BlockSpec((1, NO, D), lambda i: (i, 0, 0)),
        out_shape=jax.ShapeDtypeStruct((B, NO, D), jnp.float32),
        compiler_params=pltpu.CompilerParams(
            dimension_semantics=("parallel",)
        ),
    )(inputs)
    return lax.dynamic_update_slice(main, edges, (0, N, 0))


# speedup vs baseline: 21.9365x; 1.0321x over previous
import functools

import jax
import jax.numpy as jnp
from jax import lax
from jax.experimental import pallas as pl
from jax.experimental.pallas import tpu as pltpu
from jax.experimental.pallas import tpu_sc as plsc

B, N, D, E = 16, 2048, 256, 128
NO = N + E
NC, NS, L = 2, 16, 16
NW = NC * NS
PW = (B * E) // NW
WPB = E // PW
GW = 2 * PW

_mesh = plsc.VectorSubcoreMesh(
    core_axis_name="c", subcore_axis_name="s", num_cores=NC, num_subcores=NS
)

_GDN = lax.GatherDimensionNumbers(
    offset_dims=(), collapsed_slice_dims=(0,), start_index_map=(0,)
)


@functools.partial(
    pl.kernel,
    out_type=jax.ShapeDtypeStruct((B * E, D), jnp.float32),
    mesh=_mesh,
    scratch_types=[
        pltpu.VMEM((GW,), jnp.int32),
        pltpu.VMEM((PW,), jnp.float32),
        pltpu.VMEM((GW, D), jnp.float32),
        pltpu.VMEM((PW, D), jnp.float32),
        pltpu.SemaphoreType.DMA,
    ],
)
def _sc_edge_lerp(table, gidx, pos, out, g_v, p_v, r_v, o_v, sem):
    wid = lax.axis_index("s") * NC + lax.axis_index("c")
    base = wid * PW

    pltpu.sync_copy(gidx.at[wid], g_v)
    pltpu.sync_copy(pos.at[pl.ds((wid % WPB) * PW, PW)], p_v)
    pltpu.async_copy(table.at[g_v], r_v, sem).wait()

    def chunk(c, carry):
        pc = p_v[pl.ds(c * L, L)]

        def row(j2, carry2):
            j = c * L + j2
            pj = lax.gather(pc, jnp.full((L, 1), j2, jnp.int32), _GDN,
                            slice_sizes=(1,),
                            mode=lax.GatherScatterMode.PROMISE_IN_BOUNDS)
            qj = 1.0 - pj

            def col(k, carry3):
                sl = pl.ds(k * L, L)
                o_v[j, sl] = r_v[j, sl] * pj + r_v[PW + j, sl] * qj
                return carry3

            return lax.fori_loop(0, D // L, col, carry2)

        return lax.fori_loop(0, L, row, carry)

    lax.fori_loop(0, PW // L, chunk, 0)
    pltpu.sync_copy(o_v, out.at[pl.ds(base, PW)])


def _tc_copy_body(in_ref, out_ref):
    out_ref[:, :N, :] = in_ref[...]


def kernel(inputs, new_pts_pos, unpool_idx):
    idx = unpool_idx.astype(jnp.int32)
    table = inputs.reshape(B * N, D)
    boff = (jnp.arange(NW, dtype=jnp.int32) // WPB * N)[:, None]
    eidx = idx.T.reshape(2, WPB, PW)
    per_worker = jnp.concatenate(
        [eidx[0], eidx[1]], axis=1
    ).reshape(1, WPB, GW)
    gidx = (boff + jnp.tile(per_worker, (B, 1, 1)).reshape(NW, GW))
    edges = _sc_edge_lerp(table, gidx, new_pts_pos)
    edges = edges.reshape(B, E, D)
    main = pl.pallas_call(
        _tc_copy_body,
        grid=(B,),
        in_specs=[pl.BlockSpec((1, N, D), lambda i: (i, 0, 0))],
        out_specs=pl.BlockSpec((1, NO, D), lambda i: (i, 0, 0)),
        out_shape=jax.ShapeDtypeStruct((B, NO, D), jnp.float32),
        compiler_params=pltpu.CompilerParams(
            dimension_semantics=("parallel",)
        ),
    )(inputs)
    return lax.dynamic_update_slice(main, edges, (0, N, 0))
